# baseline (speedup 1.0000x reference)
import functools

import jax
import jax.numpy as jnp
from jax import lax
from jax.experimental import pallas as pl
from jax.experimental.pallas import tpu as pltpu
from jax.experimental.pallas import tpu_sc as plsc

EMB = 32
HIDDEN = 128
CLASS = 100
BATCH = 4096
SEQ = 200

NW = 32
BPW = BATCH // NW
LANES = 16
HALF = EMB // LANES
NBUF = 4


def _pool_body(x_hbm, table_hbm, out_hbm, xraw_v, idx_v, bufs, acc_v, sems):
    w = lax.axis_index("c") * 16 + lax.axis_index("s")
    pltpu.sync_copy(x_hbm.at[pl.ds(w * BPW, BPW), :], xraw_v)

    lane = lax.iota(jnp.int32, LANES)

    def tbody(j, carry):
        col = jnp.full((LANES,), j, jnp.int32)
        for g in range(BPW // LANES):
            vals = plsc.load_gather(xraw_v, [lane + (g * LANES), col])
            idx_v[j, pl.ds(g * LANES, LANES)] = vals
        return carry

    lax.fori_loop(0, SEQ, tbody, 0)

    zero = jnp.zeros((LANES,), jnp.float32)
    for i in range(BPW):
        for h in range(HALF):
            acc_v[i, h * LANES:(h + 1) * LANES] = zero

    def start_gather(j, buf, sem):
        pltpu.make_async_copy(table_hbm.at[idx_v.at[j]], buf, sem).start()

    def wait_gather(j, buf, sem):
        pltpu.make_async_copy(table_hbm.at[idx_v.at[j]], buf, sem).wait()

    def accumulate(buf):
        for i in range(BPW):
            for h in range(HALF):
                plsc.addupdate(
                    acc_v.at[i, pl.ds(h * LANES, LANES)],
                    buf[i, h * LANES:(h + 1) * LANES],
                )

    for k in range(NBUF):
        start_gather(k, bufs[k], sems[k])

    def body(t, carry):
        j0 = NBUF * t
        for k in range(NBUF):
            wait_gather(j0 + k, bufs[k], sems[k])
            accumulate(bufs[k])
            start_gather(j0 + k + NBUF, bufs[k], sems[k])
        return carry

    lax.fori_loop(0, SEQ // NBUF - 1, body, 0)

    for k in range(NBUF):
        wait_gather(SEQ - NBUF + k, bufs[k], sems[k])
        accumulate(bufs[k])

    pltpu.sync_copy(acc_v, out_hbm.at[pl.ds(w * BPW, BPW), :])


_pool_call = functools.partial(
    pl.kernel,
    mesh=plsc.VectorSubcoreMesh(core_axis_name="c", subcore_axis_name="s"),
    out_type=jax.ShapeDtypeStruct((BATCH, EMB), jnp.float32),
    scratch_types=[
        pltpu.VMEM((BPW, SEQ), jnp.int32),
        pltpu.VMEM((SEQ, BPW), jnp.int32),
        [pltpu.VMEM((BPW, EMB), jnp.float32) for _ in range(NBUF)],
        pltpu.VMEM((BPW, EMB), jnp.float32),
        [pltpu.SemaphoreType.DMA for _ in range(NBUF)],
    ],
    compiler_params=pltpu.CompilerParams(
        use_tc_tiling_on_sc=False, needs_layout_passes=False
    ),
)(_pool_body)


def _mlp_body(p_ref, w1_ref, b1_ref, w2_ref, b2_ref, o_ref):
    h = jnp.dot(p_ref[:], w1_ref[:], preferred_element_type=jnp.float32)
    h = jnp.maximum(h + b1_ref[:], 0.0)
    o_ref[:] = jnp.dot(h, w2_ref[:], preferred_element_type=jnp.float32) + b2_ref[:]


def _mlp_call(pooled, w1, b1, w2, b2):
    return pl.pallas_call(
        _mlp_body,
        out_shape=jax.ShapeDtypeStruct((BATCH, HIDDEN), jnp.float32),
    )(pooled, w1, b1, w2, b2)


@jax.jit
def kernel(x, table, W1, b1, W2, b2):
    pooled_sum = _pool_call(x.astype(jnp.int32), table)
    w1s = W1 * (1.0 / SEQ)
    b1r = b1.reshape(1, HIDDEN)
    w2p = jnp.pad(W2, ((0, 0), (0, HIDDEN - CLASS)))
    b2p = jnp.pad(b2, (0, HIDDEN - CLASS)).reshape(1, HIDDEN)
    out = _mlp_call(pooled_sum, w1s, b1r, w2p, b2p)
    return out[:, :CLASS]


# speedup vs baseline: 1.0207x; 1.0207x over previous
import functools

import jax
import jax.numpy as jnp
from jax import lax
from jax.experimental import pallas as pl
from jax.experimental.pallas import tpu as pltpu
from jax.experimental.pallas import tpu_sc as plsc

VROWS = 1000001
EMB = 32
HIDDEN = 128
CLASS = 100
BATCH = 4096
SEQ = 200

NW = 32
BPW = BATCH // NW
LANES = 16
HALF = EMB // LANES
NBUF = 4


def _pool_body(xw_hbm, table_hbm, out_hbm, idx_v, bufs, acc_v, sems):
    w = lax.axis_index("c") * 16 + lax.axis_index("s")
    pltpu.sync_copy(xw_hbm.at[w], idx_v)

    zero = jnp.zeros((LANES,), jnp.float32)
    for s in range(BPW * EMB // LANES):
        acc_v[pl.ds(s * LANES, LANES)] = zero

    def start_gather(j, buf, sem):
        pltpu.make_async_copy(table_hbm.at[idx_v.at[j]], buf, sem).start()

    def wait_gather(j, buf, sem):
        pltpu.make_async_copy(table_hbm.at[idx_v.at[j]], buf, sem).wait()

    def accumulate(buf):
        for i in range(BPW):
            for h in range(HALF):
                plsc.addupdate(
                    acc_v.at[pl.ds(i * EMB + h * LANES, LANES)],
                    buf[i, h * LANES:(h + 1) * LANES],
                )

    for k in range(NBUF):
        start_gather(k, bufs[k], sems[k])

    def body(t, carry):
        j0 = NBUF * t
        for k in range(NBUF):
            wait_gather(j0 + k, bufs[k], sems[k])
            accumulate(bufs[k])
            start_gather(j0 + k + NBUF, bufs[k], sems[k])
        return carry

    lax.fori_loop(0, SEQ // NBUF - 1, body, 0)

    for k in range(NBUF):
        wait_gather(SEQ - NBUF + k, bufs[k], sems[k])
        accumulate(bufs[k])

    pltpu.sync_copy(acc_v, out_hbm.at[pl.ds(w * BPW * EMB, BPW * EMB)])


_pool_call = functools.partial(
    pl.kernel,
    mesh=plsc.VectorSubcoreMesh(core_axis_name="c", subcore_axis_name="s"),
    out_type=jax.ShapeDtypeStruct((BATCH * EMB,), jnp.float32),
    scratch_types=[
        pltpu.VMEM((SEQ, BPW), jnp.int32),
        [pltpu.VMEM((BPW, HIDDEN), jnp.float32) for _ in range(NBUF)],
        pltpu.VMEM((BPW * EMB,), jnp.float32),
        [pltpu.SemaphoreType.DMA for _ in range(NBUF)],
    ],
    compiler_params=pltpu.CompilerParams(use_tc_tiling_on_sc=True),
)(_pool_body)


def _mlp_body(p_ref, w1_ref, b1_ref, w2_ref, b2_ref, o_ref):
    h = jnp.dot(p_ref[:], w1_ref[:], preferred_element_type=jnp.float32)
    h = jnp.maximum(h + b1_ref[:], 0.0)
    o_ref[:] = jnp.dot(h, w2_ref[:], preferred_element_type=jnp.float32) + b2_ref[:]


def _mlp_call(pooled, w1, b1, w2, b2):
    return pl.pallas_call(
        _mlp_body,
        out_shape=jax.ShapeDtypeStruct((BATCH, HIDDEN), jnp.float32),
    )(pooled, w1, b1, w2, b2)


@jax.jit
def kernel(x, table, W1, b1, W2, b2):
    xw = x.astype(jnp.int32).T.reshape(SEQ, NW, BPW).transpose(1, 0, 2)
    tpad = jnp.pad(table, ((0, 0), (0, HIDDEN - EMB)))
    pooled_sum = _pool_call(xw, tpad).reshape(BATCH, EMB)
    w1s = W1 * (1.0 / SEQ)
    b1r = b1.reshape(1, HIDDEN)
    w2p = jnp.pad(W2, ((0, 0), (0, HIDDEN - CLASS)))
    b2p = jnp.pad(b2, (0, HIDDEN - CLASS)).reshape(1, HIDDEN)
    out = _mlp_call(pooled_sum, w1s, b1r, w2p, b2p)
    return out[:, :CLASS]


# speedup vs baseline: 1.0467x; 1.0255x over previous
import functools

import jax
import jax.numpy as jnp
from jax import lax
from jax.experimental import pallas as pl
from jax.experimental.pallas import tpu as pltpu
from jax.experimental.pallas import tpu_sc as plsc

VOCABP = 1000001
EMB = 32
HIDDEN = 128
CLASS = 100
BATCH = 4096
SEQ = 200

NW = 32
BPW = BATCH // NW
LANES = 16
HALF = EMB // LANES
NBUF = 4

PBLK = 2048
NPBLK = (VOCABP + PBLK - 1) // PBLK


def _pool_body(xw_hbm, table_hbm, out_hbm, idx_v, bufs, acc_v, sems):
    w = lax.axis_index("c") * 16 + lax.axis_index("s")
    pltpu.sync_copy(xw_hbm.at[w], idx_v)

    zero = jnp.zeros((LANES,), jnp.float32)
    for s in range(BPW * EMB // LANES):
        acc_v[pl.ds(s * LANES, LANES)] = zero

    def start_gather(j, buf, sem):
        pltpu.make_async_copy(table_hbm.at[idx_v.at[j]], buf, sem).start()

    def wait_gather(j, buf, sem):
        pltpu.make_async_copy(table_hbm.at[idx_v.at[j]], buf, sem).wait()

    def accumulate(buf):
        for i in range(BPW):
            for h in range(HALF):
                plsc.addupdate(
                    acc_v.at[pl.ds(i * EMB + h * LANES, LANES)],
                    buf[i, h * LANES:(h + 1) * LANES],
                )

    for k in range(NBUF):
        start_gather(k, bufs[k], sems[k])

    def body(t, carry):
        j0 = NBUF * t
        for k in range(NBUF):
            wait_gather(j0 + k, bufs[k], sems[k])
            accumulate(bufs[k])
            start_gather(j0 + k + NBUF, bufs[k], sems[k])
        return carry

    lax.fori_loop(0, SEQ // NBUF - 1, body, 0)

    for k in range(NBUF):
        wait_gather(SEQ - NBUF + k, bufs[k], sems[k])
        accumulate(bufs[k])

    pltpu.sync_copy(acc_v, out_hbm.at[pl.ds(w * BPW * EMB, BPW * EMB)])


_pool_call = functools.partial(
    pl.kernel,
    mesh=plsc.VectorSubcoreMesh(core_axis_name="c", subcore_axis_name="s"),
    out_type=jax.ShapeDtypeStruct((BATCH * EMB,), jnp.float32),
    scratch_types=[
        pltpu.VMEM((SEQ, BPW), jnp.int32),
        [pltpu.VMEM((BPW, HIDDEN), jnp.float32) for _ in range(NBUF)],
        pltpu.VMEM((BPW * EMB,), jnp.float32),
        [pltpu.SemaphoreType.DMA for _ in range(NBUF)],
    ],
    compiler_params=pltpu.CompilerParams(use_tc_tiling_on_sc=True),
)(_pool_body)


def _pack_body(t_ref, eye_ref, o_ref):
    b = lax.dot_general(
        t_ref[:], eye_ref[:], (((0,), (0,)), ((), ())),
        preferred_element_type=jnp.float32)
    o_ref[:, 0:EMB] = b
    o_ref[:, EMB:HIDDEN] = jnp.zeros((PBLK, HIDDEN - EMB), jnp.float32)


def _pack_call(table_t, eye):
    return pl.pallas_call(
        _pack_body,
        grid=(NPBLK,),
        in_specs=[
            pl.BlockSpec((EMB, PBLK), lambda i: (0, i)),
            pl.BlockSpec((EMB, EMB), lambda i: (0, 0)),
        ],
        out_specs=pl.BlockSpec((PBLK, HIDDEN), lambda i: (i, 0)),
        out_shape=jax.ShapeDtypeStruct((VOCABP, HIDDEN), jnp.float32),
    )(table_t, eye)


def _mlp_body(p_ref, w1_ref, b1_ref, w2_ref, b2_ref, o_ref):
    h = jnp.dot(p_ref[:], w1_ref[:], preferred_element_type=jnp.float32)
    h = jnp.maximum(h + b1_ref[:], 0.0)
    o_ref[:] = jnp.dot(h, w2_ref[:], preferred_element_type=jnp.float32) + b2_ref[:]


def _mlp_call(pooled, w1, b1, w2, b2):
    return pl.pallas_call(
        _mlp_body,
        out_shape=jax.ShapeDtypeStruct((BATCH, HIDDEN), jnp.float32),
    )(pooled, w1, b1, w2, b2)


@jax.jit
def kernel(x, table, W1, b1, W2, b2):
    xw = x.astype(jnp.int32).T.reshape(SEQ, NW, BPW).transpose(1, 0, 2)
    tpad = _pack_call(table.T, jnp.eye(EMB, dtype=jnp.float32))
    pooled = _pool_call(xw, tpad).reshape(BATCH, EMB)
    w1s = W1 * (1.0 / SEQ)
    b1r = b1.reshape(1, HIDDEN)
    w2p = jnp.pad(W2, ((0, 0), (0, HIDDEN - CLASS)))
    b2p = jnp.pad(b2, (0, HIDDEN - CLASS)).reshape(1, HIDDEN)
    out = _mlp_call(pooled, w1s, b1r, w2p, b2p)
    return out[:, :CLASS]


# speedup vs baseline: 1.4398x; 1.3756x over previous
import functools

import jax
import jax.numpy as jnp
from jax import lax
from jax.experimental import pallas as pl
from jax.experimental.pallas import tpu as pltpu
from jax.experimental.pallas import tpu_sc as plsc

VOCABP = 1000001
EMB = 32
HIDDEN = 128
CLASS = 100
BATCH = 4096
SEQ = 200

NW = 32
BPW = BATCH // NW
LANES = 16
HALF = EMB // LANES
NBUF = 4

PBLK = 8192
NPBLK = (VOCABP + PBLK - 1) // PBLK


def _pool_body(xw_hbm, table_hbm, out_hbm, idx_v, bufs, acc_v, sems):
    w = lax.axis_index("c") * 16 + lax.axis_index("s")
    pltpu.sync_copy(xw_hbm.at[w], idx_v)

    zero = jnp.zeros((LANES,), jnp.float32)
    for s in range(BPW * EMB // LANES):
        acc_v[pl.ds(s * LANES, LANES)] = zero

    def start_gather(j, buf, sem):
        pltpu.make_async_copy(table_hbm.at[idx_v.at[j]], buf, sem).start()

    def wait_gather(j, buf, sem):
        pltpu.make_async_copy(table_hbm.at[idx_v.at[j]], buf, sem).wait()

    def accumulate(buf):
        for i in range(BPW):
            for h in range(HALF):
                plsc.addupdate(
                    acc_v.at[pl.ds(i * EMB + h * LANES, LANES)],
                    buf[i, h * LANES:(h + 1) * LANES],
                )

    for k in range(NBUF):
        start_gather(k, bufs[k], sems[k])

    def body(t, carry):
        j0 = NBUF * t
        for k in range(NBUF):
            wait_gather(j0 + k, bufs[k], sems[k])
            accumulate(bufs[k])
            start_gather(j0 + k + NBUF, bufs[k], sems[k])
        return carry

    lax.fori_loop(0, SEQ // NBUF - 1, body, 0)

    for k in range(NBUF):
        wait_gather(SEQ - NBUF + k, bufs[k], sems[k])
        accumulate(bufs[k])

    pltpu.sync_copy(acc_v, out_hbm.at[pl.ds(w * BPW * EMB, BPW * EMB)])


_pool_call = functools.partial(
    pl.kernel,
    mesh=plsc.VectorSubcoreMesh(core_axis_name="c", subcore_axis_name="s"),
    out_type=jax.ShapeDtypeStruct((BATCH * EMB,), jnp.float32),
    scratch_types=[
        pltpu.VMEM((SEQ, BPW), jnp.int32),
        [pltpu.VMEM((BPW, HIDDEN), jnp.float32) for _ in range(NBUF)],
        pltpu.VMEM((BPW * EMB,), jnp.float32),
        [pltpu.SemaphoreType.DMA for _ in range(NBUF)],
    ],
    compiler_params=pltpu.CompilerParams(use_tc_tiling_on_sc=True),
)(_pool_body)


def _pack_body(t_ref, eye_ref, o_ref):
    b = lax.dot_general(
        t_ref[:], eye_ref[:], (((0,), (0,)), ((), ())),
        preferred_element_type=jnp.float32)
    o_ref[:, 0:EMB] = b
    o_ref[:, EMB:HIDDEN] = jnp.zeros((PBLK, HIDDEN - EMB), jnp.float32)


def _pack_call(table_t, eye):
    return pl.pallas_call(
        _pack_body,
        grid=(NPBLK,),
        in_specs=[
            pl.BlockSpec((EMB, PBLK), lambda i: (0, i)),
            pl.BlockSpec((EMB, EMB), lambda i: (0, 0)),
        ],
        out_specs=pl.BlockSpec((PBLK, HIDDEN), lambda i: (i, 0)),
        out_shape=jax.ShapeDtypeStruct((VOCABP, HIDDEN), jnp.float32),
    )(table_t, eye)


def _mlp_body(p_ref, w1_ref, b1_ref, w2_ref, b2_ref, o_ref):
    h = jnp.dot(p_ref[:], w1_ref[:], preferred_element_type=jnp.float32)
    h = jnp.maximum(h + b1_ref[:], 0.0)
    o_ref[:] = jnp.dot(h, w2_ref[:], preferred_element_type=jnp.float32) + b2_ref[:]


def _mlp_call(pooled, w1, b1, w2, b2):
    return pl.pallas_call(
        _mlp_body,
        out_shape=jax.ShapeDtypeStruct((BATCH, HIDDEN), jnp.float32),
    )(pooled, w1, b1, w2, b2)


@jax.jit
def kernel(x, table, W1, b1, W2, b2):
    xw = x.astype(jnp.int32).T.reshape(SEQ, NW, BPW).transpose(1, 0, 2)
    tpad = _pack_call(table.T, jnp.eye(EMB, dtype=jnp.float32))
    pooled = _pool_call(xw, tpad).reshape(BATCH, EMB)
    w1s = W1 * (1.0 / SEQ)
    b1r = b1.reshape(1, HIDDEN)
    w2p = jnp.pad(W2, ((0, 0), (0, HIDDEN - CLASS)))
    b2p = jnp.pad(b2, (0, HIDDEN - CLASS)).reshape(1, HIDDEN)
    out = _mlp_call(pooled, w1s, b1r, w2p, b2p)
    return out[:, :CLASS]


# speedup vs baseline: 1.5613x; 1.0844x over previous
import functools

import jax
import jax.numpy as jnp
from jax import lax
from jax.experimental import pallas as pl
from jax.experimental.pallas import tpu as pltpu
from jax.experimental.pallas import tpu_sc as plsc

VOCABP = 1000001
EMB = 32
HIDDEN = 128
CLASS = 100
BATCH = 4096
SEQ = 200

NW = 32
BPW = BATCH // NW
LANES = 16
HALF = EMB // LANES
NBUF = 4

PBLK = 32768
NPBLK = (VOCABP + PBLK - 1) // PBLK


def _pool_body(xw_hbm, table_hbm, out_hbm, idx_v, bufs, acc_v, sems):
    w = lax.axis_index("c") * 16 + lax.axis_index("s")
    pltpu.sync_copy(xw_hbm.at[w], idx_v)

    zero = jnp.zeros((LANES,), jnp.float32)
    for s in range(BPW * EMB // LANES):
        acc_v[pl.ds(s * LANES, LANES)] = zero

    def start_gather(j, buf, sem):
        pltpu.make_async_copy(table_hbm.at[idx_v.at[j]], buf, sem).start()

    def wait_gather(j, buf, sem):
        pltpu.make_async_copy(table_hbm.at[idx_v.at[j]], buf, sem).wait()

    def accumulate(buf):
        for i in range(BPW):
            for h in range(HALF):
                plsc.addupdate(
                    acc_v.at[pl.ds(i * EMB + h * LANES, LANES)],
                    buf[i, h * LANES:(h + 1) * LANES],
                )

    for k in range(NBUF):
        start_gather(k, bufs[k], sems[k])

    def body(t, carry):
        j0 = NBUF * t
        for k in range(NBUF):
            wait_gather(j0 + k, bufs[k], sems[k])
            accumulate(bufs[k])
            start_gather(j0 + k + NBUF, bufs[k], sems[k])
        return carry

    lax.fori_loop(0, SEQ // NBUF - 1, body, 0)

    for k in range(NBUF):
        wait_gather(SEQ - NBUF + k, bufs[k], sems[k])
        accumulate(bufs[k])

    pltpu.sync_copy(acc_v, out_hbm.at[pl.ds(w * BPW * EMB, BPW * EMB)])


_pool_call = functools.partial(
    pl.kernel,
    mesh=plsc.VectorSubcoreMesh(core_axis_name="c", subcore_axis_name="s"),
    out_type=jax.ShapeDtypeStruct((BATCH * EMB,), jnp.float32),
    scratch_types=[
        pltpu.VMEM((SEQ, BPW), jnp.int32),
        [pltpu.VMEM((BPW, HIDDEN), jnp.float32) for _ in range(NBUF)],
        pltpu.VMEM((BPW * EMB,), jnp.float32),
        [pltpu.SemaphoreType.DMA for _ in range(NBUF)],
    ],
    compiler_params=pltpu.CompilerParams(use_tc_tiling_on_sc=True),
)(_pool_body)


def _pack_body(t_ref, eye_ref, o_ref):
    b = lax.dot_general(
        t_ref[:], eye_ref[:], (((0,), (0,)), ((), ())),
        preferred_element_type=jnp.float32)
    o_ref[:, 0:EMB] = b
    o_ref[:, EMB:HIDDEN] = jnp.zeros((PBLK, HIDDEN - EMB), jnp.float32)


def _pack_call(table_t, eye):
    return pl.pallas_call(
        _pack_body,
        grid=(NPBLK,),
        in_specs=[
            pl.BlockSpec((EMB, PBLK), lambda i: (0, i)),
            pl.BlockSpec((EMB, EMB), lambda i: (0, 0)),
        ],
        out_specs=pl.BlockSpec((PBLK, HIDDEN), lambda i: (i, 0)),
        out_shape=jax.ShapeDtypeStruct((VOCABP, HIDDEN), jnp.float32),
    )(table_t, eye)


def _mlp_body(p_ref, w1_ref, b1_ref, w2_ref, b2_ref, o_ref):
    h = jnp.dot(p_ref[:], w1_ref[:], preferred_element_type=jnp.float32)
    h = jnp.maximum(h + b1_ref[:], 0.0)
    o_ref[:] = jnp.dot(h, w2_ref[:], preferred_element_type=jnp.float32) + b2_ref[:]


def _mlp_call(pooled, w1, b1, w2, b2):
    return pl.pallas_call(
        _mlp_body,
        out_shape=jax.ShapeDtypeStruct((BATCH, HIDDEN), jnp.float32),
    )(pooled, w1, b1, w2, b2)


@jax.jit
def kernel(x, table, W1, b1, W2, b2):
    xw = x.astype(jnp.int32).T.reshape(SEQ, NW, BPW).transpose(1, 0, 2)
    tpad = _pack_call(table.T, jnp.eye(EMB, dtype=jnp.float32))
    pooled = _pool_call(xw, tpad).reshape(BATCH, EMB)
    w1s = W1 * (1.0 / SEQ)
    b1r = b1.reshape(1, HIDDEN)
    w2p = jnp.pad(W2, ((0, 0), (0, HIDDEN - CLASS)))
    b2p = jnp.pad(b2, (0, HIDDEN - CLASS)).reshape(1, HIDDEN)
    out = _mlp_call(pooled, w1s, b1r, w2p, b2p)
    return out[:, :CLASS]


# speedup vs baseline: 1.5648x; 1.0023x over previous
import functools

import jax
import jax.numpy as jnp
from jax import lax
from jax.experimental import pallas as pl
from jax.experimental.pallas import tpu as pltpu
from jax.experimental.pallas import tpu_sc as plsc

VOCABP = 1000001
EMB = 32
HIDDEN = 128
CLASS = 100
BATCH = 4096
SEQ = 200

NW = 32
BPW = BATCH // NW
LANES = 16
HALF = EMB // LANES
NBUF = 4

PBLK = 32768
NPBLK = (VOCABP + PBLK - 1) // PBLK


def _pool_body(xw_hbm, table_hbm, out_hbm, idx_v, bufs, acc_v, sems):
    w = lax.axis_index("c") * 16 + lax.axis_index("s")
    pltpu.sync_copy(xw_hbm.at[w], idx_v)

    zero = jnp.zeros((LANES,), jnp.float32)
    for s in range(BPW * EMB // LANES):
        acc_v[pl.ds(s * LANES, LANES)] = zero

    def _copies(j, buf, sem):
        return [
            pltpu.make_async_copy(
                table_hbm.at[idx_v.at[j, pl.ds(hh * 64, 64)]],
                buf.at[pl.ds(hh * 64, 64), :],
                sem[hh],
            )
            for hh in range(2)
        ]

    def start_gather(j, buf, sem):
        for c in _copies(j, buf, sem):
            c.start()

    def wait_gather(j, buf, sem):
        for c in _copies(j, buf, sem):
            c.wait()

    def accumulate(buf):
        for i in range(BPW):
            for h in range(HALF):
                plsc.addupdate(
                    acc_v.at[pl.ds(i * EMB + h * LANES, LANES)],
                    buf[i, h * LANES:(h + 1) * LANES],
                )

    for k in range(NBUF):
        start_gather(k, bufs[k], sems[k])

    def body(t, carry):
        j0 = NBUF * t
        for k in range(NBUF):
            wait_gather(j0 + k, bufs[k], sems[k])
            accumulate(bufs[k])
            start_gather(j0 + k + NBUF, bufs[k], sems[k])
        return carry

    lax.fori_loop(0, SEQ // NBUF - 1, body, 0)

    for k in range(NBUF):
        wait_gather(SEQ - NBUF + k, bufs[k], sems[k])
        accumulate(bufs[k])

    pltpu.sync_copy(acc_v, out_hbm.at[pl.ds(w * BPW * EMB, BPW * EMB)])


_pool_call = functools.partial(
    pl.kernel,
    mesh=plsc.VectorSubcoreMesh(core_axis_name="c", subcore_axis_name="s"),
    out_type=jax.ShapeDtypeStruct((BATCH * EMB,), jnp.float32),
    scratch_types=[
        pltpu.VMEM((SEQ, BPW), jnp.int32),
        [pltpu.VMEM((BPW, HIDDEN), jnp.float32) for _ in range(NBUF)],
        pltpu.VMEM((BPW * EMB,), jnp.float32),
        [[pltpu.SemaphoreType.DMA, pltpu.SemaphoreType.DMA] for _ in range(NBUF)],
    ],
    compiler_params=pltpu.CompilerParams(use_tc_tiling_on_sc=True),
)(_pool_body)


def _pack_body(t_ref, eye_ref, o_ref):
    b = lax.dot_general(
        t_ref[:], eye_ref[:], (((0,), (0,)), ((), ())),
        preferred_element_type=jnp.float32)
    o_ref[:, 0:EMB] = b
    o_ref[:, EMB:HIDDEN] = jnp.zeros((PBLK, HIDDEN - EMB), jnp.float32)


def _pack_call(table_t, eye):
    return pl.pallas_call(
        _pack_body,
        grid=(NPBLK,),
        in_specs=[
            pl.BlockSpec((EMB, PBLK), lambda i: (0, i)),
            pl.BlockSpec((EMB, EMB), lambda i: (0, 0)),
        ],
        out_specs=pl.BlockSpec((PBLK, HIDDEN), lambda i: (i, 0)),
        out_shape=jax.ShapeDtypeStruct((VOCABP, HIDDEN), jnp.float32),
    )(table_t, eye)


def _mlp_body(p_ref, w1_ref, b1_ref, w2_ref, b2_ref, o_ref):
    h = jnp.dot(p_ref[:], w1_ref[:], preferred_element_type=jnp.float32)
    h = jnp.maximum(h + b1_ref[:], 0.0)
    o_ref[:] = jnp.dot(h, w2_ref[:], preferred_element_type=jnp.float32) + b2_ref[:]


def _mlp_call(pooled, w1, b1, w2, b2):
    return pl.pallas_call(
        _mlp_body,
        out_shape=jax.ShapeDtypeStruct((BATCH, HIDDEN), jnp.float32),
    )(pooled, w1, b1, w2, b2)


@jax.jit
def kernel(x, table, W1, b1, W2, b2):
    xw = x.astype(jnp.int32).T.reshape(SEQ, NW, BPW).transpose(1, 0, 2)
    tpad = _pack_call(table.T, jnp.eye(EMB, dtype=jnp.float32))
    pooled = _pool_call(xw, tpad).reshape(BATCH, EMB)
    w1s = W1 * (1.0 / SEQ)
    b1r = b1.reshape(1, HIDDEN)
    w2p = jnp.pad(W2, ((0, 0), (0, HIDDEN - CLASS)))
    b2p = jnp.pad(b2, (0, HIDDEN - CLASS)).reshape(1, HIDDEN)
    out = _mlp_call(pooled, w1s, b1r, w2p, b2p)
    return out[:, :CLASS]
